# baseline (speedup 1.0000x reference)
import functools
import jax
import jax.numpy as jnp
from jax import lax
from jax.experimental import pallas as pl
from jax.experimental.pallas import tpu as pltpu
from jax.experimental.pallas import tpu_sc as plsc

B, S, H = 4, 4096, 1024
L = 16
SLICES = H // L
NW = 32
SEQ_PER_W = S // NW
C = 32
N_SEQ_CHUNKS = SEQ_PER_W // C
N_CHUNKS = N_SEQ_CHUNKS * B
GROUPS = C // L
RI = 8


def _rsqrt_v(x):
    i = plsc.bitcast(x, jnp.int32)
    i = jnp.int32(0x5F3759DF) - (i >> 1)
    y = plsc.bitcast(i, jnp.float32)
    half = x * 0.5
    for _ in range(4):
        y = y * (1.5 - half * y * y)
    return y


def _make_sc_kernel():
    mesh = plsc.VectorSubcoreMesh(core_axis_name="c", subcore_axis_name="s")

    @functools.partial(
        pl.kernel,
        mesh=mesh,
        out_type=jax.ShapeDtypeStruct((B * S, H), jnp.float32),
        scratch_types=[
            pltpu.VMEM((C, H), jnp.float32),
            pltpu.VMEM((C, H), jnp.float32),
            pltpu.VMEM((C, H), jnp.float32),
            pltpu.VMEM((H,), jnp.float32),
            pltpu.VMEM((H,), jnp.float32),
            pltpu.VMEM((C, L), jnp.float32),
            pltpu.VMEM((C, L), jnp.float32),
            pltpu.VMEM((GROUPS, L), jnp.float32),
            pltpu.VMEM((GROUPS, L), jnp.float32),
            pltpu.SemaphoreType.DMA,
            pltpu.SemaphoreType.DMA,
            pltpu.SemaphoreType.DMA,
            pltpu.SemaphoreType.DMA,
            pltpu.SemaphoreType.DMA,
        ],
        compiler_params=pltpu.CompilerParams(needs_layout_passes=False),
    )
    def sc_kernel(x_hbm, pos_hbm, gamma_hbm, beta_hbm, out_hbm,
                  xbuf0, xbuf1, posbuf, gbuf, bbuf, sbuf, qbuf, mbuf, ibuf,
                  ld0, ld1, st0, st1, pld):
        wid = lax.axis_index("s") * 2 + lax.axis_index("c")
        seq_w0 = wid * SEQ_PER_W
        lanes = lax.broadcasted_iota(jnp.int32, (L,), 0)
        xbufs = (xbuf0, xbuf1)
        lds = (ld0, ld1)
        sts = (st0, st1)

        pltpu.sync_copy(gamma_hbm, gbuf)
        pltpu.sync_copy(beta_hbm, bbuf)

        def compute(xb):
            def row_pass1(r, _):
                zero = jnp.zeros((L,), jnp.float32)

                def sl(i, carry):
                    acc = list(carry)
                    for u in range(4):
                        off = (i + u) * L
                        e = xb[r, pl.ds(off, L)] + posbuf[r, pl.ds(off, L)]
                        xb[r, pl.ds(off, L)] = e
                        acc[2 * u] = acc[2 * u] + e
                        acc[2 * u + 1] = acc[2 * u + 1] + e * e
                    return tuple(acc)

                acc = plsc.parallel_loop(
                    0, SLICES, 4, unroll=2, carry=(zero,) * 8)(sl)
                sbuf[r, :] = acc[0] + acc[2] + acc[4] + acc[6]
                qbuf[r, :] = acc[1] + acc[3] + acc[5] + acc[7]
                return 0

            lax.fori_loop(0, C, row_pass1, 0, unroll=2)

            for g in range(GROUPS):
                rows_idx = g * L + lanes
                tot_s = jnp.zeros((L,), jnp.float32)
                tot_q = jnp.zeros((L,), jnp.float32)
                for lcol in range(L):
                    ci_v = jnp.full((L,), lcol, jnp.int32)
                    tot_s = tot_s + plsc.load_gather(sbuf, [rows_idx, ci_v])
                    tot_q = tot_q + plsc.load_gather(qbuf, [rows_idx, ci_v])
                mean_v = tot_s * (1.0 / H)
                var_v = tot_q * (1.0 / H) - mean_v * mean_v
                mbuf[g, :] = mean_v
                ibuf[g, :] = _rsqrt_v(var_v + 1e-6)

            def p2_block(rb, _):
                vmeans = []
                vinvs = []
                for j in range(RI):
                    r = rb * RI + j
                    gi = jnp.full((L,), r // L, jnp.int32)
                    ri = jnp.full((L,), r % L, jnp.int32)
                    vmeans.append(plsc.load_gather(mbuf, [gi, ri]))
                    vinvs.append(plsc.load_gather(ibuf, [gi, ri]))

                def sl(i):
                    off = i * L
                    gg = gbuf[pl.ds(off, L)]
                    bb = bbuf[pl.ds(off, L)]
                    for j in range(RI):
                        r = rb * RI + j
                        e = xb[r, pl.ds(off, L)]
                        xb[r, pl.ds(off, L)] = \
                            (e - vmeans[j]) * vinvs[j] * gg + bb

                plsc.parallel_loop(0, SLICES, 1, unroll=2)(sl)
                return 0

            lax.fori_loop(0, C // RI, p2_block, 0)

        def row0_of(ci):
            sc_i, b = divmod(ci, B)
            return b * S + seq_w0 + sc_i * C

        pos_cp = pltpu.async_copy(pos_hbm.at[pl.ds(seq_w0, C)], posbuf, pld)
        x_cp = [None, None]
        st_cp = [None, None]
        x_cp[0] = pltpu.async_copy(
            x_hbm.at[pl.ds(row0_of(0), C)], xbufs[0], lds[0])

        for ci in range(N_CHUNKS):
            sc_i, b = divmod(ci, B)
            par = ci % 2
            if ci + 1 < N_CHUNKS:
                npar = (ci + 1) % 2
                if ci >= 1:
                    st_cp[npar].wait()
                x_cp[npar] = pltpu.async_copy(
                    x_hbm.at[pl.ds(row0_of(ci + 1), C)], xbufs[npar],
                    lds[npar])
            x_cp[par].wait()
            if b == 0:
                pos_cp.wait()
            compute(xbufs[par])
            if b == B - 1 and sc_i + 1 < N_SEQ_CHUNKS:
                pos_cp = pltpu.async_copy(
                    pos_hbm.at[pl.ds(seq_w0 + (sc_i + 1) * C, C)], posbuf,
                    pld)
            st_cp[par] = pltpu.async_copy(
                xbufs[par], out_hbm.at[pl.ds(row0_of(ci), C)], sts[par])

        st_cp[0].wait()
        st_cp[1].wait()

    return sc_kernel


_SC_KERNEL = _make_sc_kernel()


def kernel(input_ids, pos_emb, gamma, beta):
    x2 = input_ids.reshape(B * S, H)
    out = _SC_KERNEL(x2, pos_emb, gamma, beta)
    return out.reshape(B, S, H)


# speedup vs baseline: 1.0074x; 1.0074x over previous
import functools
import jax
import jax.numpy as jnp
from jax import lax
from jax.experimental import pallas as pl
from jax.experimental.pallas import tpu as pltpu
from jax.experimental.pallas import tpu_sc as plsc

B, S, H = 4, 4096, 1024
L = 16
SLICES = H // L
NW = 32
SEQ_PER_W = S // NW
C = 32
N_SEQ_CHUNKS = SEQ_PER_W // C
N_CHUNKS = N_SEQ_CHUNKS * B
GROUPS = C // L
RI = 8


def _rsqrt_v(x):
    i = plsc.bitcast(x, jnp.int32)
    i = jnp.int32(0x5F3759DF) - (i >> 1)
    y = plsc.bitcast(i, jnp.float32)
    half = x * 0.5
    for _ in range(4):
        y = y * (1.5 - half * y * y)
    return y


def _make_sc_kernel():
    mesh = plsc.VectorSubcoreMesh(core_axis_name="c", subcore_axis_name="s")

    @functools.partial(
        pl.kernel,
        mesh=mesh,
        out_type=jax.ShapeDtypeStruct((B * S, H), jnp.float32),
        scratch_types=[
            pltpu.VMEM((C, H), jnp.float32),
            pltpu.VMEM((C, H), jnp.float32),
            pltpu.VMEM((C, H), jnp.float32),
            pltpu.VMEM((H,), jnp.float32),
            pltpu.VMEM((H,), jnp.float32),
            pltpu.VMEM((C, L), jnp.float32),
            pltpu.VMEM((C, L), jnp.float32),
            pltpu.VMEM((GROUPS, L), jnp.float32),
            pltpu.VMEM((GROUPS, L), jnp.float32),
            pltpu.SemaphoreType.DMA,
            pltpu.SemaphoreType.DMA,
            pltpu.SemaphoreType.DMA,
            pltpu.SemaphoreType.DMA,
            pltpu.SemaphoreType.DMA,
        ],
        compiler_params=pltpu.CompilerParams(needs_layout_passes=False),
    )
    def sc_kernel(x_hbm, pos_hbm, gamma_hbm, beta_hbm, out_hbm,
                  xbuf0, xbuf1, posbuf, gbuf, bbuf, sbuf, qbuf, mbuf, ibuf,
                  ld0, ld1, st0, st1, pld):
        wid = lax.axis_index("s") * 2 + lax.axis_index("c")
        seq_w0 = wid * SEQ_PER_W
        lanes = lax.broadcasted_iota(jnp.int32, (L,), 0)
        xbufs = (xbuf0, xbuf1)
        lds = (ld0, ld1)
        sts = (st0, st1)

        pltpu.sync_copy(gamma_hbm, gbuf)
        pltpu.sync_copy(beta_hbm, bbuf)

        def compute(xb):
            def row_pass1(r, _):
                zero = jnp.zeros((L,), jnp.float32)

                def sl(i, carry):
                    a0, q0, a1, q1 = carry
                    off = i * L
                    e0 = xb[r, pl.ds(off, L)] + posbuf[r, pl.ds(off, L)]
                    xb[r, pl.ds(off, L)] = e0
                    off1 = off + L
                    e1 = xb[r, pl.ds(off1, L)] + posbuf[r, pl.ds(off1, L)]
                    xb[r, pl.ds(off1, L)] = e1
                    return (a0 + e0, q0 + e0 * e0, a1 + e1, q1 + e1 * e1)

                a0, q0, a1, q1 = plsc.parallel_loop(
                    0, SLICES, 2, unroll=8, carry=(zero, zero, zero, zero))(sl)
                sbuf[r, :] = a0 + a1
                qbuf[r, :] = q0 + q1
                return 0

            lax.fori_loop(0, C, row_pass1, 0, unroll=2)

            for g in range(GROUPS):
                rows_idx = g * L + lanes
                tot_s = jnp.zeros((L,), jnp.float32)
                tot_q = jnp.zeros((L,), jnp.float32)
                for lcol in range(L):
                    ci_v = jnp.full((L,), lcol, jnp.int32)
                    tot_s = tot_s + plsc.load_gather(sbuf, [rows_idx, ci_v])
                    tot_q = tot_q + plsc.load_gather(qbuf, [rows_idx, ci_v])
                mean_v = tot_s * (1.0 / H)
                var_v = tot_q * (1.0 / H) - mean_v * mean_v
                mbuf[g, :] = mean_v
                ibuf[g, :] = _rsqrt_v(var_v + 1e-6)

            def p2_block(rb, _):
                vmeans = []
                vinvs = []
                for j in range(RI):
                    r = rb * RI + j
                    gi = jnp.full((L,), r // L, jnp.int32)
                    ri = jnp.full((L,), r % L, jnp.int32)
                    vmeans.append(plsc.load_gather(mbuf, [gi, ri]))
                    vinvs.append(plsc.load_gather(ibuf, [gi, ri]))

                def sl(i):
                    off = i * L
                    gg = gbuf[pl.ds(off, L)]
                    bb = bbuf[pl.ds(off, L)]
                    for j in range(RI):
                        r = rb * RI + j
                        e = xb[r, pl.ds(off, L)]
                        xb[r, pl.ds(off, L)] = \
                            (e - vmeans[j]) * vinvs[j] * gg + bb

                plsc.parallel_loop(0, SLICES, 1, unroll=2)(sl)
                return 0

            lax.fori_loop(0, C // RI, p2_block, 0)

        def row0_of(ci):
            sc_i, b = divmod(ci, B)
            return b * S + seq_w0 + sc_i * C

        pos_cp = pltpu.async_copy(pos_hbm.at[pl.ds(seq_w0, C)], posbuf, pld)
        x_cp = [None, None]
        st_cp = [None, None]
        x_cp[0] = pltpu.async_copy(
            x_hbm.at[pl.ds(row0_of(0), C)], xbufs[0], lds[0])

        for ci in range(N_CHUNKS):
            sc_i, b = divmod(ci, B)
            par = ci % 2
            if ci + 1 < N_CHUNKS:
                npar = (ci + 1) % 2
                if ci >= 1:
                    st_cp[npar].wait()
                x_cp[npar] = pltpu.async_copy(
                    x_hbm.at[pl.ds(row0_of(ci + 1), C)], xbufs[npar],
                    lds[npar])
            x_cp[par].wait()
            if b == 0:
                pos_cp.wait()
            compute(xbufs[par])
            if b == B - 1 and sc_i + 1 < N_SEQ_CHUNKS:
                pos_cp = pltpu.async_copy(
                    pos_hbm.at[pl.ds(seq_w0 + (sc_i + 1) * C, C)], posbuf,
                    pld)
            st_cp[par] = pltpu.async_copy(
                xbufs[par], out_hbm.at[pl.ds(row0_of(ci), C)], sts[par])

        st_cp[0].wait()
        st_cp[1].wait()

    return sc_kernel


_SC_KERNEL = _make_sc_kernel()


def kernel(input_ids, pos_emb, gamma, beta):
    x2 = input_ids.reshape(B * S, H)
    out = _SC_KERNEL(x2, pos_emb, gamma, beta)
    return out.reshape(B, S, H)


# speedup vs baseline: 1.0137x; 1.0063x over previous
import functools
import jax
import jax.numpy as jnp
from jax import lax
from jax.experimental import pallas as pl
from jax.experimental.pallas import tpu as pltpu
from jax.experimental.pallas import tpu_sc as plsc

B, S, H = 4, 4096, 1024
L = 16
SLICES = H // L
NW = 32
SEQ_PER_W = S // NW
C = 32
N_SEQ_CHUNKS = SEQ_PER_W // C
N_CHUNKS = N_SEQ_CHUNKS * B
GROUPS = C // L
RI = 16


def _rsqrt_v(x):
    i = plsc.bitcast(x, jnp.int32)
    i = jnp.int32(0x5F3759DF) - (i >> 1)
    y = plsc.bitcast(i, jnp.float32)
    half = x * 0.5
    for _ in range(4):
        y = y * (1.5 - half * y * y)
    return y


def _make_sc_kernel():
    mesh = plsc.VectorSubcoreMesh(core_axis_name="c", subcore_axis_name="s")

    @functools.partial(
        pl.kernel,
        mesh=mesh,
        out_type=jax.ShapeDtypeStruct((B * S, H), jnp.float32),
        scratch_types=[
            pltpu.VMEM((C, H), jnp.float32),
            pltpu.VMEM((C, H), jnp.float32),
            pltpu.VMEM((C, H), jnp.float32),
            pltpu.VMEM((H,), jnp.float32),
            pltpu.VMEM((H,), jnp.float32),
            pltpu.VMEM((C, L), jnp.float32),
            pltpu.VMEM((C, L), jnp.float32),
            pltpu.VMEM((GROUPS, L), jnp.float32),
            pltpu.VMEM((GROUPS, L), jnp.float32),
            pltpu.SemaphoreType.DMA,
            pltpu.SemaphoreType.DMA,
            pltpu.SemaphoreType.DMA,
            pltpu.SemaphoreType.DMA,
            pltpu.SemaphoreType.DMA,
        ],
        compiler_params=pltpu.CompilerParams(needs_layout_passes=False),
    )
    def sc_kernel(x_hbm, pos_hbm, gamma_hbm, beta_hbm, out_hbm,
                  xbuf0, xbuf1, posbuf, gbuf, bbuf, sbuf, qbuf, mbuf, ibuf,
                  ld0, ld1, st0, st1, pld):
        wid = lax.axis_index("s") * 2 + lax.axis_index("c")
        seq_w0 = wid * SEQ_PER_W
        lanes = lax.broadcasted_iota(jnp.int32, (L,), 0)
        xbufs = (xbuf0, xbuf1)
        lds = (ld0, ld1)
        sts = (st0, st1)

        pltpu.sync_copy(gamma_hbm, gbuf)
        pltpu.sync_copy(beta_hbm, bbuf)

        def compute(xb):
            def row_pass1(r, _):
                zero = jnp.zeros((L,), jnp.float32)

                def sl(i, carry):
                    a0, q0, a1, q1 = carry
                    off = i * L
                    e0 = xb[r, pl.ds(off, L)] + posbuf[r, pl.ds(off, L)]
                    xb[r, pl.ds(off, L)] = e0
                    off1 = off + L
                    e1 = xb[r, pl.ds(off1, L)] + posbuf[r, pl.ds(off1, L)]
                    xb[r, pl.ds(off1, L)] = e1
                    return (a0 + e0, q0 + e0 * e0, a1 + e1, q1 + e1 * e1)

                a0, q0, a1, q1 = plsc.parallel_loop(
                    0, SLICES, 2, unroll=4, carry=(zero, zero, zero, zero))(sl)
                sbuf[r, :] = a0 + a1
                qbuf[r, :] = q0 + q1
                return 0

            lax.fori_loop(0, C, row_pass1, 0, unroll=2)

            for g in range(GROUPS):
                rows_idx = g * L + lanes
                tot_s = jnp.zeros((L,), jnp.float32)
                tot_q = jnp.zeros((L,), jnp.float32)
                for lcol in range(L):
                    ci_v = jnp.full((L,), lcol, jnp.int32)
                    tot_s = tot_s + plsc.load_gather(sbuf, [rows_idx, ci_v])
                    tot_q = tot_q + plsc.load_gather(qbuf, [rows_idx, ci_v])
                mean_v = tot_s * (1.0 / H)
                var_v = tot_q * (1.0 / H) - mean_v * mean_v
                mbuf[g, :] = mean_v
                ibuf[g, :] = _rsqrt_v(var_v + 1e-6)

            def p2_block(rb, _):
                vmeans = []
                vinvs = []
                for j in range(RI):
                    r = rb * RI + j
                    gi = jnp.full((L,), r // L, jnp.int32)
                    ri = jnp.full((L,), r % L, jnp.int32)
                    vmeans.append(plsc.load_gather(mbuf, [gi, ri]))
                    vinvs.append(plsc.load_gather(ibuf, [gi, ri]))

                def sl(i):
                    off = i * L
                    gg = gbuf[pl.ds(off, L)]
                    bb = bbuf[pl.ds(off, L)]
                    for j in range(RI):
                        r = rb * RI + j
                        e = xb[r, pl.ds(off, L)]
                        xb[r, pl.ds(off, L)] = \
                            (e - vmeans[j]) * vinvs[j] * gg + bb

                plsc.parallel_loop(0, SLICES, 1, unroll=1)(sl)
                return 0

            lax.fori_loop(0, C // RI, p2_block, 0)

        def row0_of(ci):
            sc_i, b = divmod(ci, B)
            return b * S + seq_w0 + sc_i * C

        pos_cp = pltpu.async_copy(pos_hbm.at[pl.ds(seq_w0, C)], posbuf, pld)
        x_cp = [None, None]
        st_cp = [None, None]
        x_cp[0] = pltpu.async_copy(
            x_hbm.at[pl.ds(row0_of(0), C)], xbufs[0], lds[0])

        for ci in range(N_CHUNKS):
            sc_i, b = divmod(ci, B)
            par = ci % 2
            if ci + 1 < N_CHUNKS:
                npar = (ci + 1) % 2
                if ci >= 1:
                    st_cp[npar].wait()
                x_cp[npar] = pltpu.async_copy(
                    x_hbm.at[pl.ds(row0_of(ci + 1), C)], xbufs[npar],
                    lds[npar])
            x_cp[par].wait()
            if b == 0:
                pos_cp.wait()
            compute(xbufs[par])
            if b == B - 1 and sc_i + 1 < N_SEQ_CHUNKS:
                pos_cp = pltpu.async_copy(
                    pos_hbm.at[pl.ds(seq_w0 + (sc_i + 1) * C, C)], posbuf,
                    pld)
            st_cp[par] = pltpu.async_copy(
                xbufs[par], out_hbm.at[pl.ds(row0_of(ci), C)], sts[par])

        st_cp[0].wait()
        st_cp[1].wait()

    return sc_kernel


_SC_KERNEL = _make_sc_kernel()


def kernel(input_ids, pos_emb, gamma, beta):
    x2 = input_ids.reshape(B * S, H)
    out = _SC_KERNEL(x2, pos_emb, gamma, beta)
    return out.reshape(B, S, H)


# speedup vs baseline: 1.0260x; 1.0121x over previous
import functools
import jax
import jax.numpy as jnp
from jax import lax
from jax.experimental import pallas as pl
from jax.experimental.pallas import tpu as pltpu
from jax.experimental.pallas import tpu_sc as plsc

B, S, H = 4, 4096, 1024
L = 16
SLICES = H // L
NW = 32
SEQ_PER_W = S // NW
C = 32
N_SEQ_CHUNKS = SEQ_PER_W // C
N_CHUNKS = N_SEQ_CHUNKS * B
GROUPS = C // L
RI = 8


def _rsqrt_v(x):
    i = plsc.bitcast(x, jnp.int32)
    i = jnp.int32(0x5F3759DF) - (i >> 1)
    y = plsc.bitcast(i, jnp.float32)
    half = x * 0.5
    for _ in range(4):
        y = y * (1.5 - half * y * y)
    return y


def _make_sc_kernel():
    mesh = plsc.VectorSubcoreMesh(core_axis_name="c", subcore_axis_name="s")

    @functools.partial(
        pl.kernel,
        mesh=mesh,
        out_type=jax.ShapeDtypeStruct((B * S, H), jnp.float32),
        scratch_types=[
            pltpu.VMEM((C, H), jnp.float32),
            pltpu.VMEM((C, H), jnp.float32),
            pltpu.VMEM((C, H), jnp.float32),
            pltpu.VMEM((H,), jnp.float32),
            pltpu.VMEM((H,), jnp.float32),
            pltpu.VMEM((C, L), jnp.float32),
            pltpu.VMEM((C, L), jnp.float32),
            pltpu.VMEM((GROUPS, L), jnp.float32),
            pltpu.VMEM((GROUPS, L), jnp.float32),
            pltpu.SemaphoreType.DMA,
            pltpu.SemaphoreType.DMA,
            pltpu.SemaphoreType.DMA,
            pltpu.SemaphoreType.DMA,
            pltpu.SemaphoreType.DMA,
        ],
        compiler_params=pltpu.CompilerParams(needs_layout_passes=False),
    )
    def sc_kernel(x_hbm, pos_hbm, gamma_hbm, beta_hbm, out_hbm,
                  xbuf0, xbuf1, posbuf, gbuf, bbuf, sbuf, qbuf, mbuf, ibuf,
                  ld0, ld1, st0, st1, pld):
        wid = lax.axis_index("s") * 2 + lax.axis_index("c")
        seq_w0 = wid * SEQ_PER_W
        lanes = lax.broadcasted_iota(jnp.int32, (L,), 0)
        xbufs = (xbuf0, xbuf1)
        lds = (ld0, ld1)
        sts = (st0, st1)

        pltpu.sync_copy(gamma_hbm, gbuf)
        pltpu.sync_copy(beta_hbm, bbuf)

        def compute(xb):
            def row_pass1(r, _):
                zero = jnp.zeros((L,), jnp.float32)

                def sl(i, carry):
                    a0, q0, a1, q1 = carry
                    off = i * L
                    e0 = xb[r, pl.ds(off, L)] + posbuf[r, pl.ds(off, L)]
                    xb[r, pl.ds(off, L)] = e0
                    off1 = off + L
                    e1 = xb[r, pl.ds(off1, L)] + posbuf[r, pl.ds(off1, L)]
                    xb[r, pl.ds(off1, L)] = e1
                    return (a0 + e0, q0 + e0 * e0, a1 + e1, q1 + e1 * e1)

                a0, q0, a1, q1 = plsc.parallel_loop(
                    0, SLICES, 2, unroll=4, carry=(zero, zero, zero, zero))(sl)
                sbuf[r, :] = a0 + a1
                qbuf[r, :] = q0 + q1
                return 0

            lax.fori_loop(0, C, row_pass1, 0, unroll=2)

            for g in range(GROUPS):
                rows_idx = g * L + lanes
                tot_s = jnp.zeros((L,), jnp.float32)
                tot_q = jnp.zeros((L,), jnp.float32)
                for lcol in range(L):
                    ci_v = jnp.full((L,), lcol, jnp.int32)
                    tot_s = tot_s + plsc.load_gather(sbuf, [rows_idx, ci_v])
                    tot_q = tot_q + plsc.load_gather(qbuf, [rows_idx, ci_v])
                mean_v = tot_s * (1.0 / H)
                var_v = tot_q * (1.0 / H) - mean_v * mean_v
                mbuf[g, :] = mean_v
                ibuf[g, :] = _rsqrt_v(var_v + 1e-6)

            def p2_block(rb, _):
                vmeans = []
                vinvs = []
                for j in range(RI):
                    r = rb * RI + j
                    gi = jnp.full((L,), r // L, jnp.int32)
                    ri = jnp.full((L,), r % L, jnp.int32)
                    vmeans.append(plsc.load_gather(mbuf, [gi, ri]))
                    vinvs.append(plsc.load_gather(ibuf, [gi, ri]))

                def sl(i):
                    off = i * L
                    gg = gbuf[pl.ds(off, L)]
                    bb = bbuf[pl.ds(off, L)]
                    for j in range(RI):
                        r = rb * RI + j
                        e = xb[r, pl.ds(off, L)]
                        xb[r, pl.ds(off, L)] = \
                            (e - vmeans[j]) * vinvs[j] * gg + bb

                plsc.parallel_loop(0, SLICES, 1, unroll=2)(sl)
                return 0

            lax.fori_loop(0, C // RI, p2_block, 0)

        def row0_of(ci):
            sc_i, b = divmod(ci, B)
            return b * S + seq_w0 + sc_i * C

        pos_cp = pltpu.async_copy(pos_hbm.at[pl.ds(seq_w0, C)], posbuf, pld)
        x_cp = [None, None]
        st_cp = [None, None]
        x_cp[0] = pltpu.async_copy(
            x_hbm.at[pl.ds(row0_of(0), C)], xbufs[0], lds[0])

        for ci in range(N_CHUNKS):
            sc_i, b = divmod(ci, B)
            par = ci % 2
            if ci + 1 < N_CHUNKS:
                npar = (ci + 1) % 2
                if ci >= 1:
                    st_cp[npar].wait()
                x_cp[npar] = pltpu.async_copy(
                    x_hbm.at[pl.ds(row0_of(ci + 1), C)], xbufs[npar],
                    lds[npar])
            x_cp[par].wait()
            if b == 0:
                pos_cp.wait()
            compute(xbufs[par])
            if b == B - 1 and sc_i + 1 < N_SEQ_CHUNKS:
                pos_cp = pltpu.async_copy(
                    pos_hbm.at[pl.ds(seq_w0 + (sc_i + 1) * C, C)], posbuf,
                    pld)
            st_cp[par] = pltpu.async_copy(
                xbufs[par], out_hbm.at[pl.ds(row0_of(ci), C)], sts[par])

        st_cp[0].wait()
        st_cp[1].wait()

    return sc_kernel


_SC_KERNEL = _make_sc_kernel()


def kernel(input_ids, pos_emb, gamma, beta):
    x2 = input_ids.reshape(B * S, H)
    out = _SC_KERNEL(x2, pos_emb, gamma, beta)
    return out.reshape(B, S, H)
